# baseline (speedup 1.0000x reference)
import functools

import jax
import jax.numpy as jnp
from jax import lax
from jax.experimental import pallas as pl
from jax.experimental.pallas import tpu as pltpu
from jax.experimental.pallas import tpu_sc as plsc

NC, NS = 2, 16
NW = NC * NS
D = 32
VOCAB = 26
L = 16
GROUP = 1024

BATCH = 16384
SEQ = 200
N = BATCH * SEQ
ROWS_PER_W = N // NW
GROUPS_PER_W = ROWS_PER_W // GROUP

_mesh = plsc.VectorSubcoreMesh(core_axis_name="c", subcore_axis_name="s")


@functools.partial(
    pl.kernel,
    out_type=jax.ShapeDtypeStruct((N * D,), jnp.float32),
    mesh=_mesh,
    scratch_types=[
        pltpu.VMEM((VOCAB * D,), jnp.float32),
        pltpu.VMEM((2 * GROUP,), jnp.int32),
        pltpu.VMEM((2 * GROUP * D,), jnp.float32),
        pltpu.SemaphoreType.DMA,
        pltpu.SemaphoreType.DMA,
        pltpu.SemaphoreType.DMA,
        pltpu.SemaphoreType.DMA,
    ],
    compiler_params=pltpu.CompilerParams(
        use_tc_tiling_on_sc=False, needs_layout_passes=False),
)
def _gather_kernel(idx_hbm, table_hbm, out_hbm, table_v, idx_v, rows_v,
                   i_sem0, i_sem1, o_sem0, o_sem1):
    wid = lax.axis_index("s") * NC + lax.axis_index("c")
    idx0 = wid * ROWS_PER_W
    i_sems = (i_sem0, i_sem1)
    o_sems = (o_sem0, o_sem1)

    pltpu.sync_copy(table_hbm, table_v)

    lane_rows = lax.iota(jnp.int32, L) * D

    def idx_copy(g, buf):
        return pltpu.make_async_copy(
            idx_hbm.at[pl.ds(idx0 + g * GROUP, GROUP)],
            idx_v.at[pl.ds(buf * GROUP, GROUP)],
            i_sems[buf])

    def out_copy(g, buf):
        return pltpu.make_async_copy(
            rows_v.at[pl.ds(buf * GROUP * D, GROUP * D)],
            out_hbm.at[pl.ds((idx0 + g * GROUP) * D, GROUP * D)],
            o_sems[buf])

    def phase(g, buf):
        @pl.when(g + 1 < GROUPS_PER_W)
        def _prefetch():
            idx_copy(g + 1, 1 - buf).start()

        idx_copy(g, buf).wait()

        @pl.when(g >= 2)
        def _reuse():
            out_copy(g - 2, buf).wait()

        def chunk(c, carry):
            off = buf * GROUP + c * L
            i16 = idx_v[pl.ds(off, L)]
            src = i16 * D
            dst = lane_rows + off * D
            for j in range(D):
                vals = plsc.load_gather(table_v, [src + j])
                plsc.store_scatter(rows_v, [dst + j], vals)
            return carry

        lax.fori_loop(0, GROUP // L, chunk, 0)

        out_copy(g, buf).start()

    idx_copy(0, 0).start()

    def pair(t, carry):
        phase(2 * t, 0)
        phase(2 * t + 1, 1)
        return carry

    lax.fori_loop(0, GROUPS_PER_W // 2, pair, 0)

    out_copy(GROUPS_PER_W - 2, 0).wait()
    out_copy(GROUPS_PER_W - 1, 1).wait()


def kernel(protein_1d_data, embedding_table):
    idx = protein_1d_data.astype(jnp.int32).reshape(N)
    out = _gather_kernel(idx, embedding_table.reshape(VOCAB * D))
    return out.reshape(BATCH, SEQ, D)


# speedup vs baseline: 1.2126x; 1.2126x over previous
import functools

import jax
import jax.numpy as jnp
from jax import lax
from jax.experimental import pallas as pl
from jax.experimental.pallas import tpu as pltpu
from jax.experimental.pallas import tpu_sc as plsc

NC, NS = 2, 16
NW = NC * NS
D = 32
VOCAB = 26
L = 16
GROUP = 1024

BATCH = 16384
SEQ = 200
N = BATCH * SEQ
ROWS_PER_W = N // NW
GROUPS_PER_W = ROWS_PER_W // GROUP

_mesh = plsc.VectorSubcoreMesh(core_axis_name="c", subcore_axis_name="s")


@functools.partial(
    pl.kernel,
    out_type=jax.ShapeDtypeStruct((N * D,), jnp.float32),
    mesh=_mesh,
    scratch_types=[
        pltpu.VMEM((VOCAB * D,), jnp.float32),
        pltpu.VMEM((2 * GROUP,), jnp.int32),
        pltpu.VMEM((2 * GROUP * D,), jnp.float32),
        pltpu.SemaphoreType.DMA,
        pltpu.SemaphoreType.DMA,
        pltpu.SemaphoreType.DMA,
        pltpu.SemaphoreType.DMA,
    ],
    compiler_params=pltpu.CompilerParams(
        use_tc_tiling_on_sc=False, needs_layout_passes=False),
)
def _gather_kernel(idx_hbm, table_hbm, out_hbm, table_v, idx_v, rows_v,
                   i_sem0, i_sem1, o_sem0, o_sem1):
    wid = lax.axis_index("s") * NC + lax.axis_index("c")
    idx0 = wid * ROWS_PER_W
    i_sems = (i_sem0, i_sem1)
    o_sems = (o_sem0, o_sem1)

    pltpu.sync_copy(table_hbm, table_v)

    lane_rows = lax.iota(jnp.int32, L) * D

    def idx_copy(g, buf):
        return pltpu.make_async_copy(
            idx_hbm.at[pl.ds(idx0 + g * GROUP, GROUP)],
            idx_v.at[pl.ds(buf * GROUP, GROUP)],
            i_sems[buf])

    def out_copy(g, buf):
        return pltpu.make_async_copy(
            rows_v.at[pl.ds(buf * GROUP * D, GROUP * D)],
            out_hbm.at[pl.ds((idx0 + g * GROUP) * D, GROUP * D)],
            o_sems[buf])

    def phase(g, buf):
        @pl.when(g + 1 < GROUPS_PER_W)
        def _prefetch():
            idx_copy(g + 1, 1 - buf).start()

        idx_copy(g, buf).wait()

        @pl.when(g >= 2)
        def _reuse():
            out_copy(g - 2, buf).wait()

        @plsc.parallel_loop(0, GROUP // L, 1, unroll=4)
        def chunk(c):
            off = buf * GROUP + c * L
            i16 = idx_v[pl.ds(off, L)]
            src = i16 * D
            dst = lane_rows + off * D
            for j in range(D):
                vals = plsc.load_gather(table_v, [src + j])
                plsc.store_scatter(rows_v, [dst + j], vals)

        out_copy(g, buf).start()

    idx_copy(0, 0).start()

    def pair(t, carry):
        phase(2 * t, 0)
        phase(2 * t + 1, 1)
        return carry

    lax.fori_loop(0, GROUPS_PER_W // 2, pair, 0)

    out_copy(GROUPS_PER_W - 2, 0).wait()
    out_copy(GROUPS_PER_W - 1, 1).wait()


def kernel(protein_1d_data, embedding_table):
    idx = protein_1d_data.astype(jnp.int32).reshape(N)
    out = _gather_kernel(idx, embedding_table.reshape(VOCAB * D))
    return out.reshape(BATCH, SEQ, D)


# speedup vs baseline: 2.9411x; 2.4255x over previous
import functools

import jax
import jax.numpy as jnp
from jax import lax
from jax.experimental import pallas as pl
from jax.experimental.pallas import tpu as pltpu
from jax.experimental.pallas import tpu_sc as plsc

NC, NS = 2, 16
NW = NC * NS
D = 32
VOCAB = 26
L = 16
CHUNK = 128
K = 8
GROUP = CHUNK * K

BATCH = 16384
SEQ = 200
N = BATCH * SEQ
ROWS_PER_W = N // NW
GROUPS_PER_W = ROWS_PER_W // GROUP

_mesh = plsc.VectorSubcoreMesh(core_axis_name="c", subcore_axis_name="s")


@functools.partial(
    pl.kernel,
    out_type=jax.ShapeDtypeStruct((N, D), jnp.float32),
    mesh=_mesh,
    scratch_types=[
        pltpu.VMEM_SHARED((NS * VOCAB, D), jnp.float32),
        pltpu.VMEM((2 * GROUP,), jnp.int32),
        pltpu.VMEM((2 * GROUP, D), jnp.float32),
        pltpu.SemaphoreType.DMA,
        pltpu.SemaphoreType.DMA,
        pltpu.SemaphoreType.DMA,
        pltpu.SemaphoreType.DMA,
        pltpu.SemaphoreType.DMA,
    ],
    compiler_params=pltpu.CompilerParams(
        use_tc_tiling_on_sc=False, needs_layout_passes=False),
)
def _gather_kernel(idx_hbm, table_hbm, out_hbm, table_s, idx_v, rows_v,
                   gat_sem, i_sem0, i_sem1, o_sem0, o_sem1):
    sid = lax.axis_index("s")
    wid = sid * NC + lax.axis_index("c")
    idx0 = wid * ROWS_PER_W
    i_sems = (i_sem0, i_sem1)
    o_sems = (o_sem0, o_sem1)

    pltpu.sync_copy(table_hbm, table_s.at[pl.ds(sid * VOCAB, VOCAB)])
    plsc.subcore_barrier()

    my_base = (sid * VOCAB).astype(jnp.int32)

    def idx_copy(g, buf):
        return pltpu.make_async_copy(
            idx_hbm.at[pl.ds(idx0 + g * GROUP, GROUP)],
            idx_v.at[pl.ds(buf * GROUP, GROUP)],
            i_sems[buf])

    def out_copy(g, buf):
        return pltpu.make_async_copy(
            rows_v.at[pl.ds(buf * GROUP, GROUP)],
            out_hbm.at[pl.ds(idx0 + g * GROUP, GROUP)],
            o_sems[buf])

    def phase(g, buf):
        @pl.when(g + 1 < GROUPS_PER_W)
        def _prefetch():
            idx_copy(g + 1, 1 - buf).start()

        idx_copy(g, buf).wait()

        @plsc.parallel_loop(0, GROUP // L, 1, unroll=8)
        def adjust(c):
            off = buf * GROUP + c * L
            idx_v[pl.ds(off, L)] = idx_v[pl.ds(off, L)] + my_base

        @pl.when(g >= 2)
        def _reuse():
            out_copy(g - 2, buf).wait()

        copies = [
            pltpu.make_async_copy(
                table_s.at[idx_v.at[pl.ds(buf * GROUP + j * CHUNK, CHUNK)]],
                rows_v.at[pl.ds(buf * GROUP + j * CHUNK, CHUNK)],
                gat_sem,
            )
            for j in range(K)
        ]
        for cp in copies:
            cp.start()
        for cp in copies:
            cp.wait()

        out_copy(g, buf).start()

    idx_copy(0, 0).start()

    def pair(t, carry):
        phase(2 * t, 0)
        phase(2 * t + 1, 1)
        return carry

    lax.fori_loop(0, GROUPS_PER_W // 2, pair, 0)

    out_copy(GROUPS_PER_W - 2, 0).wait()
    out_copy(GROUPS_PER_W - 1, 1).wait()


def kernel(protein_1d_data, embedding_table):
    idx = protein_1d_data.astype(jnp.int32).reshape(N)
    out = _gather_kernel(idx, embedding_table)
    return out.reshape(BATCH, SEQ, D)


# speedup vs baseline: 14.3107x; 4.8658x over previous
import functools

import jax
import jax.numpy as jnp
from jax import lax
from jax.experimental import pallas as pl
from jax.experimental.pallas import tpu as pltpu
from jax.experimental.pallas import tpu_sc as plsc

NC, NS = 2, 16
NW = NC * NS
D = 32
VOCAB = 26
VPAD = 32
L = 16

BATCH = 16384
SEQ = 200
NBT = BATCH // 128
QB = 16
NQ = NBT // QB
UNITS = SEQ * NQ
UNITS_PER_W = UNITS // NW

_mesh = plsc.VectorSubcoreMesh(core_axis_name="c", subcore_axis_name="s")


@functools.partial(
    pl.kernel,
    out_type=jax.ShapeDtypeStruct((SEQ, D // 8, NBT, 8, 128), jnp.float32),
    mesh=_mesh,
    scratch_types=[
        pltpu.VMEM((D * VPAD,), jnp.float32),
        pltpu.VMEM((2, QB, 128), jnp.int32),
        pltpu.VMEM((2, QB, 8, 128), jnp.float32),
        pltpu.SemaphoreType.DMA,
        pltpu.SemaphoreType.DMA,
        pltpu.SemaphoreType.DMA,
        pltpu.SemaphoreType.DMA,
    ],
    compiler_params=pltpu.CompilerParams(
        use_tc_tiling_on_sc=False, needs_layout_passes=False),
)
def _gather_kernel(idx_hbm, tableT_hbm, out_hbm, table_v, idx_v, obuf,
                   i_sem0, i_sem1, o_sem0, o_sem1):
    wid = lax.axis_index("s") * NC + lax.axis_index("c")
    u0 = wid * UNITS_PER_W
    i_sems = (i_sem0, i_sem1)
    o_sems = (o_sem0, o_sem1)

    pltpu.sync_copy(tableT_hbm, table_v)

    def idx_copy(u, buf):
        s = lax.shift_right_logical(u, 3)
        q = lax.bitwise_and(u, NQ - 1)
        st = lax.shift_right_logical(s, 3)
        si = lax.bitwise_and(s, 7)
        return pltpu.make_async_copy(
            idx_hbm.at[st, pl.ds(q * QB, QB), si],
            idx_v.at[buf],
            i_sems[buf])

    def out_copy(u, dt, p):
        s = lax.shift_right_logical(u, 3)
        q = lax.bitwise_and(u, NQ - 1)
        return pltpu.make_async_copy(
            obuf.at[p],
            out_hbm.at[s, dt, pl.ds(q * QB, QB)],
            o_sems[p])

    idx_copy(u0, 0).start()

    def unit(i, pi):
        u = u0 + i

        @pl.when(i + 1 < UNITS_PER_W)
        def _prefetch():
            idx_copy(u + 1, 1 - pi).start()

        idx_copy(u, pi).wait()

        for dt in range(4):
            p = dt & 1

            if dt >= 2:
                out_copy(u, dt - 2, p).wait()
            else:
                @pl.when(i > 0)
                def _w():
                    out_copy(u - 1, dt + 2, p).wait()

            @plsc.parallel_loop(0, QB, 1, unroll=1)
            def btl(t):
                for bic in range(8):
                    i16 = idx_v[pi, t, pl.ds(bic * L, L)]
                    for di in range(8):
                        g = plsc.load_gather(
                            table_v, [i16 + (dt * 8 + di) * VPAD])
                        obuf[p, t, di, pl.ds(bic * L, L)] = g

            out_copy(u, dt, p).start()

    def pair(t, carry):
        unit(2 * t, 0)
        unit(2 * t + 1, 1)
        return carry

    lax.fori_loop(0, UNITS_PER_W // 2, pair, 0)

    out_copy(u0 + UNITS_PER_W - 1, 2, 0).wait()
    out_copy(u0 + UNITS_PER_W - 1, 3, 1).wait()


def kernel(protein_1d_data, embedding_table):
    idxT = (protein_1d_data.astype(jnp.int32).T
            .reshape(SEQ // 8, 8, NBT, 128).transpose(0, 2, 1, 3))
    tableT = jnp.pad(embedding_table,
                     ((0, VPAD - VOCAB), (0, 0))).T.reshape(D * VPAD)
    out5 = _gather_kernel(idxT, tableT)
    return out5.transpose(2, 4, 0, 1, 3).reshape(BATCH, SEQ, D)
